# baseline (speedup 1.0000x reference)
import functools
import math

import jax
import jax.numpy as jnp
from jax import lax
from jax.experimental import pallas as pl
from jax.experimental.pallas import tpu as pltpu
from jax.experimental.pallas import tpu_sc as plsc

_SEQ = 4096
_BATCH = 4
_DIM = 64
_SCALE = math.sqrt(_DIM)

_NC = 2
_NS = 16
_NW = _NC * _NS
_ROWS = _SEQ * _BATCH
_RPW = _ROWS // _NW
_SPW = _SEQ // _NW
_CHUNK = 128
_NCHUNK = _RPW // _CHUNK
_LANES = 16


@functools.partial(
    pl.kernel,
    out_type=jax.ShapeDtypeStruct((_ROWS, _DIM), jnp.float32),
    mesh=plsc.VectorSubcoreMesh(core_axis_name="c", subcore_axis_name="s"),
    scratch_types=[
        pltpu.VMEM((_NCHUNK, _CHUNK), jnp.int32),
        pltpu.VMEM((_RPW, _DIM), jnp.float32),
        pltpu.VMEM((_SPW, _DIM), jnp.float32),
        pltpu.SemaphoreType.DMA,
    ],
    compiler_params=pltpu.CompilerParams(use_tc_tiling_on_sc=False),
)
def _pe_embed(idx_hbm, table_hbm, pe_hbm, out_hbm, idx_v, rows_v, pe_v, sem):
    wid = lax.axis_index("s") * _NC + lax.axis_index("c")
    base = wid * _RPW
    sbase = wid * _SPW

    pltpu.sync_copy(idx_hbm.at[pl.ds(wid * _NCHUNK, _NCHUNK)], idx_v)

    copies = []
    for j in range(_NCHUNK):
        copies.append(
            pltpu.async_copy(
                table_hbm.at[idx_v.at[j]],
                rows_v.at[pl.ds(j * _CHUNK, _CHUNK)],
                sem,
            )
        )
    pltpu.sync_copy(pe_hbm.at[pl.ds(sbase, _SPW)], pe_v)
    for cp in copies:
        cp.wait()

    def body(s, carry):
        r0 = s * _BATCH
        for c in range(_DIM // _LANES):
            pvec = pe_v[s, pl.ds(c * _LANES, _LANES)]
            for b in range(_BATCH):
                rv = rows_v[r0 + b, pl.ds(c * _LANES, _LANES)]
                rows_v[r0 + b, pl.ds(c * _LANES, _LANES)] = rv * _SCALE + pvec
        return carry

    lax.fori_loop(0, _SPW, body, 0)

    pltpu.sync_copy(rows_v, out_hbm.at[pl.ds(base, _RPW)])


def kernel(x, table, pe):
    idx2d = x.reshape(_NW * _NCHUNK, _CHUNK)
    pe2d = pe[:_SEQ, 0, :]
    out = _pe_embed(idx2d, table, pe2d)
    return out.reshape(_SEQ, _BATCH, _DIM)
